# Initial kernel scaffold; baseline (speedup 1.0000x reference)
import functools

import jax
import jax.numpy as jnp
from jax import lax
from jax.experimental import pallas as pl
from jax.experimental.pallas import tpu as pltpu
from jax.experimental.pallas import tpu_sc as plsc

N = 10000
E = 320000
D = 128
EPS = 1e-5

NC = 2
NS = 16
NW = NC * NS
K = 128
CHUNKS = 80
EPAD = NW * CHUNKS * K
NPAD = 10240
STRIPE = NPAD // NS

_mesh = plsc.VectorSubcoreMesh(core_axis_name="c", subcore_axis_name="s")


@functools.partial(
    pl.kernel,
    out_type=jax.ShapeDtypeStruct((NC, 2, NPAD), jnp.float32),
    mesh=_mesh,
    scratch_types=[
        pltpu.VMEM((CHUNKS, K), jnp.int32),
        pltpu.VMEM((CHUNKS, K), jnp.int32),
        pltpu.VMEM((K,), jnp.float32),
        pltpu.VMEM_SHARED((NPAD,), jnp.float32),
        pltpu.VMEM_SHARED((NPAD,), jnp.float32),
    ],
)
def _sc_degrees(src_hbm, dst_hbm, zrow_hbm, out_hbm, sbuf, dbuf, ones_v,
                outd_s, ind_s):
    c = lax.axis_index("c")
    s = lax.axis_index("s")
    wid = c * NS + s
    for i in range(K // 16):
        ones_v[pl.ds(i * 16, 16)] = jnp.ones((16,), jnp.float32)
    rows = pl.ds(s * STRIPE, STRIPE)
    pltpu.sync_copy(zrow_hbm.at[pl.ds(0, STRIPE)], outd_s.at[rows])
    pltpu.sync_copy(zrow_hbm.at[pl.ds(0, STRIPE)], ind_s.at[rows])
    pltpu.sync_copy(src_hbm.at[pl.ds(wid * CHUNKS, CHUNKS), :], sbuf)
    pltpu.sync_copy(dst_hbm.at[pl.ds(wid * CHUNKS, CHUNKS), :], dbuf)
    plsc.subcore_barrier()

    @pl.loop(0, CHUNKS)
    def _(i):
        pltpu.sync_copy(ones_v, outd_s.at[sbuf.at[i]], add=True)
        pltpu.sync_copy(ones_v, ind_s.at[dbuf.at[i]], add=True)

    plsc.subcore_barrier()
    pltpu.sync_copy(outd_s.at[rows], out_hbm.at[c, 0, rows])
    pltpu.sync_copy(ind_s.at[rows], out_hbm.at[c, 1, rows])


@functools.partial(
    pl.kernel,
    out_type=jax.ShapeDtypeStruct((NC, NPAD, D), jnp.float32),
    mesh=_mesh,
    scratch_types=[
        pltpu.VMEM((CHUNKS, K), jnp.int32),
        pltpu.VMEM((CHUNKS, K), jnp.int32),
        pltpu.VMEM((K, D), jnp.float32),
        pltpu.VMEM_SHARED((NPAD, D), jnp.float32),
        pltpu.SemaphoreType.DMA,
    ],
)
def _sc_aggregate(t_hbm, src_hbm, dst_hbm, zmat_hbm, out_hbm, sbuf, dbuf,
                  rows_v, acc_s, sem):
    c = lax.axis_index("c")
    s = lax.axis_index("s")
    wid = c * NS + s
    rows = pl.ds(s * STRIPE, STRIPE)
    pltpu.sync_copy(zmat_hbm.at[rows], acc_s.at[rows])
    pltpu.sync_copy(src_hbm.at[pl.ds(wid * CHUNKS, CHUNKS), :], sbuf)
    pltpu.sync_copy(dst_hbm.at[pl.ds(wid * CHUNKS, CHUNKS), :], dbuf)
    plsc.subcore_barrier()

    @pl.loop(0, CHUNKS)
    def _(i):
        pltpu.async_copy(t_hbm.at[sbuf.at[i]], rows_v, sem).wait()
        pltpu.sync_copy(rows_v, acc_s.at[dbuf.at[i]], add=True)

    plsc.subcore_barrier()
    pltpu.sync_copy(acc_s.at[rows], out_hbm.at[c, rows])


R = 512
GRID = NPAD // R


def _norms(deg_blk):
    out_deg = deg_blk[:, 0:1] + deg_blk[:, 2:3]
    in_deg = deg_blk[:, 1:2] + deg_blk[:, 3:4]
    out_norm = lax.rsqrt(jnp.maximum(out_deg, 1.0))
    in_norm = lax.rsqrt(jnp.maximum(in_deg, 1.0))
    return out_norm, in_norm


def _layer_norm_relu(x, g, be):
    mu = jnp.mean(x, axis=-1, keepdims=True)
    var = jnp.mean((x - mu) ** 2, axis=-1, keepdims=True)
    return jnp.maximum((x - mu) * lax.rsqrt(var + EPS) * g + be, 0.0)


def _tc1_body(f_ref, deg_ref, w_ref, t_ref):
    out_norm, _ = _norms(deg_ref[...])
    t_ref[...] = jnp.dot(f_ref[...] * out_norm, w_ref[...],
                         preferred_element_type=jnp.float32)


def _tc2_body(agg_ref, deg_ref, b_ref, g_ref, be_ref, w_ref, t_ref):
    out_norm, in_norm = _norms(deg_ref[...])
    x = (agg_ref[0] + agg_ref[1]) * in_norm + b_ref[...]
    h = _layer_norm_relu(x, g_ref[...], be_ref[...])
    t_ref[...] = jnp.dot(h * out_norm, w_ref[...],
                         preferred_element_type=jnp.float32)


def _tc3_body(agg_ref, deg_ref, b_ref, g_ref, be_ref, o_ref):
    _, in_norm = _norms(deg_ref[...])
    x = (agg_ref[0] + agg_ref[1]) * in_norm + b_ref[...]
    o_ref[...] = _layer_norm_relu(x, g_ref[...], be_ref[...])


_row_spec = pl.BlockSpec((R, D), lambda i: (i, 0))
_deg_spec = pl.BlockSpec((R, 4), lambda i: (i, 0))
_agg_spec = pl.BlockSpec((NC, R, D), lambda i: (0, i, 0))
_w_spec = pl.BlockSpec((D, D), lambda i: (0, 0))
_vec_spec = pl.BlockSpec((1, D), lambda i: (0, 0))

_tc1 = pl.pallas_call(
    _tc1_body,
    grid=(GRID,),
    in_specs=[_row_spec, _deg_spec, _w_spec],
    out_specs=_row_spec,
    out_shape=jax.ShapeDtypeStruct((NPAD, D), jnp.float32),
)

_tc2 = pl.pallas_call(
    _tc2_body,
    grid=(GRID,),
    in_specs=[_agg_spec, _deg_spec, _vec_spec, _vec_spec, _vec_spec, _w_spec],
    out_specs=_row_spec,
    out_shape=jax.ShapeDtypeStruct((NPAD, D), jnp.float32),
)

_tc3 = pl.pallas_call(
    _tc3_body,
    grid=(GRID,),
    in_specs=[_agg_spec, _deg_spec, _vec_spec, _vec_spec, _vec_spec],
    out_specs=_row_spec,
    out_shape=jax.ShapeDtypeStruct((NPAD, D), jnp.float32),
)


def kernel(features, edge_index, W1, b1, g1, be1, W2, b2, g2, be2):
    src = edge_index[0]
    dst = edge_index[1]
    pad = jnp.full((EPAD - E,), N, dtype=jnp.int32)
    src2d = jnp.concatenate([src, pad]).reshape(EPAD // K, K)
    dst2d = jnp.concatenate([dst, pad]).reshape(EPAD // K, K)
    fpad = jnp.pad(features, ((0, NPAD - N), (0, 0)))
    zrow = jnp.zeros((NPAD,), jnp.float32)
    zmat = jnp.zeros((NPAD, D), jnp.float32)

    degp = _sc_degrees(src2d, dst2d, zrow)
    degs = jnp.moveaxis(degp.reshape(4, NPAD), 0, 1)

    t1 = _tc1(fpad, degs, W1)
    agg1 = _sc_aggregate(t1, src2d, dst2d, zmat)
    t2 = _tc2(agg1, degs, b1.reshape(1, D), g1.reshape(1, D),
              be1.reshape(1, D), W2)
    agg2 = _sc_aggregate(t2, src2d, dst2d, zmat)
    out = _tc3(agg2, degs, b2.reshape(1, D), g2.reshape(1, D),
               be2.reshape(1, D))
    return out[:N]


# speedup vs baseline: 3.4667x; 3.4667x over previous
import functools

import jax
import jax.numpy as jnp
from jax import lax
from jax.experimental import pallas as pl
from jax.experimental.pallas import tpu as pltpu
from jax.experimental.pallas import tpu_sc as plsc

N = 10000
E = 320000
D = 128
EPS = 1e-5

NC = 2
NS = 16
NW = NC * NS
K = 128
CHUNKS = 80
EPAD = NW * CHUNKS * K
NPAD = 10240
STRIPE = NPAD // NS

_mesh = plsc.VectorSubcoreMesh(core_axis_name="c", subcore_axis_name="s",
                               num_cores=NC, num_subcores=NS)


@functools.partial(
    pl.kernel,
    out_type=jax.ShapeDtypeStruct((NC, 2, NPAD), jnp.float32),
    mesh=_mesh,
    scratch_types=[
        pltpu.VMEM((CHUNKS, K), jnp.int32),
        pltpu.VMEM((CHUNKS, K), jnp.int32),
        pltpu.VMEM((K,), jnp.float32),
        pltpu.VMEM_SHARED((NPAD,), jnp.float32),
        pltpu.VMEM_SHARED((NPAD,), jnp.float32),
    ],
)
def _sc_degrees(src_hbm, dst_hbm, zrow_hbm, out_hbm, sbuf, dbuf, ones_v,
                outd_s, ind_s):
    c = lax.axis_index("c")
    s = lax.axis_index("s")
    wid = c * NS + s
    for i in range(K // 16):
        ones_v[pl.ds(i * 16, 16)] = jnp.ones((16,), jnp.float32)
    rows = pl.ds(s * STRIPE, STRIPE)
    pltpu.sync_copy(zrow_hbm.at[pl.ds(0, STRIPE)], outd_s.at[rows])
    pltpu.sync_copy(zrow_hbm.at[pl.ds(0, STRIPE)], ind_s.at[rows])
    pltpu.sync_copy(src_hbm.at[pl.ds(wid * CHUNKS, CHUNKS), :], sbuf)
    pltpu.sync_copy(dst_hbm.at[pl.ds(wid * CHUNKS, CHUNKS), :], dbuf)
    plsc.subcore_barrier()

    @pl.loop(0, CHUNKS)
    def _(i):
        pltpu.sync_copy(ones_v, outd_s.at[sbuf.at[i]], add=True)
        pltpu.sync_copy(ones_v, ind_s.at[dbuf.at[i]], add=True)

    plsc.subcore_barrier()
    pltpu.sync_copy(outd_s.at[rows], out_hbm.at[c, 0, rows])
    pltpu.sync_copy(ind_s.at[rows], out_hbm.at[c, 1, rows])


@functools.partial(
    pl.kernel,
    out_type=jax.ShapeDtypeStruct((NC, NPAD, D), jnp.float32),
    mesh=_mesh,
    scratch_types=[
        pltpu.VMEM((CHUNKS, K), jnp.int32),
        pltpu.VMEM((CHUNKS, K), jnp.int32),
        pltpu.VMEM((K, D), jnp.float32),
        pltpu.VMEM_SHARED((NPAD, D), jnp.float32),
        pltpu.SemaphoreType.DMA,
    ],
)
def _sc_aggregate(t_hbm, src_hbm, dst_hbm, zmat_hbm, out_hbm, sbuf, dbuf,
                  rows_v, acc_s, sem):
    c = lax.axis_index("c")
    s = lax.axis_index("s")
    wid = c * NS + s
    rows = pl.ds(s * STRIPE, STRIPE)
    pltpu.sync_copy(zmat_hbm.at[rows], acc_s.at[rows])
    pltpu.sync_copy(src_hbm.at[pl.ds(wid * CHUNKS, CHUNKS), :], sbuf)
    pltpu.sync_copy(dst_hbm.at[pl.ds(wid * CHUNKS, CHUNKS), :], dbuf)
    plsc.subcore_barrier()

    @pl.loop(0, CHUNKS)
    def _(i):
        pltpu.async_copy(t_hbm.at[sbuf.at[i]], rows_v, sem).wait()
        pltpu.sync_copy(rows_v, acc_s.at[dbuf.at[i]], add=True)

    plsc.subcore_barrier()
    pltpu.sync_copy(acc_s.at[rows], out_hbm.at[c, rows])


R = 512
GRID = NPAD // R


def _norms(deg_blk):
    out_deg = deg_blk[:, 0:1] + deg_blk[:, 2:3]
    in_deg = deg_blk[:, 1:2] + deg_blk[:, 3:4]
    out_norm = lax.rsqrt(jnp.maximum(out_deg, 1.0))
    in_norm = lax.rsqrt(jnp.maximum(in_deg, 1.0))
    return out_norm, in_norm


def _layer_norm_relu(x, g, be):
    mu = jnp.mean(x, axis=-1, keepdims=True)
    var = jnp.mean((x - mu) ** 2, axis=-1, keepdims=True)
    return jnp.maximum((x - mu) * lax.rsqrt(var + EPS) * g + be, 0.0)


def _tc1_body(f_ref, deg_ref, w_ref, t_ref):
    out_norm, _ = _norms(deg_ref[...])
    t_ref[...] = jnp.dot(f_ref[...] * out_norm, w_ref[...],
                         preferred_element_type=jnp.float32)


def _tc2_body(agg_ref, deg_ref, b_ref, g_ref, be_ref, w_ref, t_ref):
    out_norm, in_norm = _norms(deg_ref[...])
    x = (agg_ref[0] + agg_ref[1]) * in_norm + b_ref[...]
    h = _layer_norm_relu(x, g_ref[...], be_ref[...])
    t_ref[...] = jnp.dot(h * out_norm, w_ref[...],
                         preferred_element_type=jnp.float32)


def _tc3_body(agg_ref, deg_ref, b_ref, g_ref, be_ref, o_ref):
    _, in_norm = _norms(deg_ref[...])
    x = (agg_ref[0] + agg_ref[1]) * in_norm + b_ref[...]
    o_ref[...] = _layer_norm_relu(x, g_ref[...], be_ref[...])


_row_spec = pl.BlockSpec((R, D), lambda i: (i, 0))
_deg_spec = pl.BlockSpec((R, 4), lambda i: (i, 0))
_agg_spec = pl.BlockSpec((NC, R, D), lambda i: (0, i, 0))
_w_spec = pl.BlockSpec((D, D), lambda i: (0, 0))
_vec_spec = pl.BlockSpec((1, D), lambda i: (0, 0))

_tc1 = pl.pallas_call(
    _tc1_body,
    grid=(GRID,),
    in_specs=[_row_spec, _deg_spec, _w_spec],
    out_specs=_row_spec,
    out_shape=jax.ShapeDtypeStruct((NPAD, D), jnp.float32),
)

_tc2 = pl.pallas_call(
    _tc2_body,
    grid=(GRID,),
    in_specs=[_agg_spec, _deg_spec, _vec_spec, _vec_spec, _vec_spec, _w_spec],
    out_specs=_row_spec,
    out_shape=jax.ShapeDtypeStruct((NPAD, D), jnp.float32),
)

_tc3 = pl.pallas_call(
    _tc3_body,
    grid=(GRID,),
    in_specs=[_agg_spec, _deg_spec, _vec_spec, _vec_spec, _vec_spec],
    out_specs=_row_spec,
    out_shape=jax.ShapeDtypeStruct((NPAD, D), jnp.float32),
)


def kernel(features, edge_index, W1, b1, g1, be1, W2, b2, g2, be2):
    src = edge_index[0]
    dst = edge_index[1]
    pad = jnp.full((EPAD - E,), N, dtype=jnp.int32)
    src2d = jnp.concatenate([src, pad]).reshape(EPAD // K, K)
    dst2d = jnp.concatenate([dst, pad]).reshape(EPAD // K, K)
    fpad = jnp.pad(features, ((0, NPAD - N), (0, 0)))
    zrow = jnp.zeros((NPAD,), jnp.float32)
    zmat = jnp.zeros((NPAD, D), jnp.float32)

    degp = _sc_degrees(src2d, dst2d, zrow)
    degs = jnp.moveaxis(degp.reshape(4, NPAD), 0, 1)

    t1 = _tc1(fpad, degs, W1)
    agg1 = _sc_aggregate(t1, src2d, dst2d, zmat)
    t2 = _tc2(agg1, degs, b1.reshape(1, D), g1.reshape(1, D),
              be1.reshape(1, D), W2)
    agg2 = _sc_aggregate(t2, src2d, dst2d, zmat)
    out = _tc3(agg2, degs, b2.reshape(1, D), g2.reshape(1, D),
               be2.reshape(1, D))
    return out[:N]


# speedup vs baseline: 3.8472x; 1.1098x over previous
import functools

import jax
import jax.numpy as jnp
from jax import lax
from jax.experimental import pallas as pl
from jax.experimental.pallas import tpu as pltpu
from jax.experimental.pallas import tpu_sc as plsc

N = 10000
E = 320000
D = 128
EPS = 1e-5

NC = 2
NS = 16
NW = NC * NS
K = 128
CHUNKS = 80
EPAD = NW * CHUNKS * K
NPAD = 10240
STRIPE = NPAD // NS

_mesh = plsc.VectorSubcoreMesh(core_axis_name="c", subcore_axis_name="s",
                               num_cores=NC, num_subcores=NS)


@functools.partial(
    pl.kernel,
    out_type=jax.ShapeDtypeStruct((NC, 2, NPAD), jnp.float32),
    mesh=_mesh,
    scratch_types=[
        pltpu.VMEM((CHUNKS, K), jnp.int32),
        pltpu.VMEM((CHUNKS, K), jnp.int32),
        pltpu.VMEM((K,), jnp.float32),
        pltpu.VMEM_SHARED((NPAD,), jnp.float32),
        pltpu.VMEM_SHARED((NPAD,), jnp.float32),
    ],
)
def _sc_degrees(src_hbm, dst_hbm, zrow_hbm, out_hbm, sbuf, dbuf, ones_v,
                outd_s, ind_s):
    c = lax.axis_index("c")
    s = lax.axis_index("s")
    wid = c * NS + s
    for i in range(K // 16):
        ones_v[pl.ds(i * 16, 16)] = jnp.ones((16,), jnp.float32)
    rows = pl.ds(s * STRIPE, STRIPE)
    pltpu.sync_copy(zrow_hbm.at[pl.ds(0, STRIPE)], outd_s.at[rows])
    pltpu.sync_copy(zrow_hbm.at[pl.ds(0, STRIPE)], ind_s.at[rows])
    pltpu.sync_copy(src_hbm.at[pl.ds(wid * CHUNKS, CHUNKS), :], sbuf)
    pltpu.sync_copy(dst_hbm.at[pl.ds(wid * CHUNKS, CHUNKS), :], dbuf)
    plsc.subcore_barrier()

    @pl.loop(0, CHUNKS)
    def _(i):
        pltpu.sync_copy(ones_v, outd_s.at[sbuf.at[i]], add=True)
        pltpu.sync_copy(ones_v, ind_s.at[dbuf.at[i]], add=True)

    plsc.subcore_barrier()
    pltpu.sync_copy(outd_s.at[rows], out_hbm.at[c, 0, rows])
    pltpu.sync_copy(ind_s.at[rows], out_hbm.at[c, 1, rows])


NB = 2
HALF = CHUNKS // 2
ROUNDS_H = HALF // NB


@functools.partial(
    pl.kernel,
    out_type=jax.ShapeDtypeStruct((NC, NPAD, D), jnp.float32),
    mesh=_mesh,
    scratch_types=[
        pltpu.VMEM((HALF, K), jnp.int32),
        pltpu.VMEM((HALF, K), jnp.int32),
        pltpu.VMEM((NB, K, D), jnp.float32),
        pltpu.VMEM_SHARED((NPAD, D), jnp.float32),
        pltpu.SemaphoreType.DMA((NB,)),
    ],
)
def _sc_aggregate(t_hbm, src_hbm, dst_hbm, zmat_hbm, out_hbm, sbuf, dbuf,
                  rows_v, acc_s, gsem):
    c = lax.axis_index("c")
    s = lax.axis_index("s")
    wid = c * NS + s
    rows = pl.ds(s * STRIPE, STRIPE)
    pltpu.sync_copy(zmat_hbm.at[rows], acc_s.at[rows])
    plsc.subcore_barrier()

    for h in range(2):
        cbase = wid * CHUNKS + h * HALF
        pltpu.sync_copy(src_hbm.at[pl.ds(cbase, HALF), :], sbuf)
        pltpu.sync_copy(dst_hbm.at[pl.ds(cbase, HALF), :], dbuf)
        for b in range(NB):
            pltpu.async_copy(t_hbm.at[sbuf.at[b]], rows_v.at[b], gsem.at[b])

        @pl.loop(0, ROUNDS_H - 1)
        def _(g):
            for b in range(NB):
                i = g * NB + b
                pltpu.make_async_copy(t_hbm.at[sbuf.at[i]], rows_v.at[b],
                                      gsem.at[b]).wait()
                pltpu.sync_copy(rows_v.at[b], acc_s.at[dbuf.at[i]], add=True)
                pltpu.async_copy(t_hbm.at[sbuf.at[i + NB]], rows_v.at[b],
                                 gsem.at[b])

        for b in range(NB):
            i = (ROUNDS_H - 1) * NB + b
            pltpu.make_async_copy(t_hbm.at[sbuf.at[i]], rows_v.at[b],
                                  gsem.at[b]).wait()
            pltpu.sync_copy(rows_v.at[b], acc_s.at[dbuf.at[i]], add=True)

    plsc.subcore_barrier()
    pltpu.sync_copy(acc_s.at[rows], out_hbm.at[c, rows])


R = 512
GRID = NPAD // R


def _norms(deg_blk):
    out_deg = deg_blk[:, 0:1] + deg_blk[:, 2:3]
    in_deg = deg_blk[:, 1:2] + deg_blk[:, 3:4]
    out_norm = lax.rsqrt(jnp.maximum(out_deg, 1.0))
    in_norm = lax.rsqrt(jnp.maximum(in_deg, 1.0))
    return out_norm, in_norm


def _layer_norm_relu(x, g, be):
    mu = jnp.mean(x, axis=-1, keepdims=True)
    var = jnp.mean((x - mu) ** 2, axis=-1, keepdims=True)
    return jnp.maximum((x - mu) * lax.rsqrt(var + EPS) * g + be, 0.0)


def _tc1_body(f_ref, deg_ref, w_ref, t_ref):
    out_norm, _ = _norms(deg_ref[...])
    t_ref[...] = jnp.dot(f_ref[...] * out_norm, w_ref[...],
                         preferred_element_type=jnp.float32)


def _tc2_body(agg_ref, deg_ref, b_ref, g_ref, be_ref, w_ref, t_ref):
    out_norm, in_norm = _norms(deg_ref[...])
    x = (agg_ref[0] + agg_ref[1]) * in_norm + b_ref[...]
    h = _layer_norm_relu(x, g_ref[...], be_ref[...])
    t_ref[...] = jnp.dot(h * out_norm, w_ref[...],
                         preferred_element_type=jnp.float32)


def _tc3_body(agg_ref, deg_ref, b_ref, g_ref, be_ref, o_ref):
    _, in_norm = _norms(deg_ref[...])
    x = (agg_ref[0] + agg_ref[1]) * in_norm + b_ref[...]
    o_ref[...] = _layer_norm_relu(x, g_ref[...], be_ref[...])


_row_spec = pl.BlockSpec((R, D), lambda i: (i, 0))
_deg_spec = pl.BlockSpec((R, 4), lambda i: (i, 0))
_agg_spec = pl.BlockSpec((NC, R, D), lambda i: (0, i, 0))
_w_spec = pl.BlockSpec((D, D), lambda i: (0, 0))
_vec_spec = pl.BlockSpec((1, D), lambda i: (0, 0))

_tc1 = pl.pallas_call(
    _tc1_body,
    grid=(GRID,),
    in_specs=[_row_spec, _deg_spec, _w_spec],
    out_specs=_row_spec,
    out_shape=jax.ShapeDtypeStruct((NPAD, D), jnp.float32),
)

_tc2 = pl.pallas_call(
    _tc2_body,
    grid=(GRID,),
    in_specs=[_agg_spec, _deg_spec, _vec_spec, _vec_spec, _vec_spec, _w_spec],
    out_specs=_row_spec,
    out_shape=jax.ShapeDtypeStruct((NPAD, D), jnp.float32),
)

_tc3 = pl.pallas_call(
    _tc3_body,
    grid=(GRID,),
    in_specs=[_agg_spec, _deg_spec, _vec_spec, _vec_spec, _vec_spec],
    out_specs=_row_spec,
    out_shape=jax.ShapeDtypeStruct((NPAD, D), jnp.float32),
)


def kernel(features, edge_index, W1, b1, g1, be1, W2, b2, g2, be2):
    src = edge_index[0]
    dst = edge_index[1]
    pad = jnp.full((EPAD - E,), N, dtype=jnp.int32)
    src2d = jnp.concatenate([src, pad]).reshape(EPAD // K, K)
    dst2d = jnp.concatenate([dst, pad]).reshape(EPAD // K, K)
    fpad = jnp.pad(features, ((0, NPAD - N), (0, 0)))
    zrow = jnp.zeros((NPAD,), jnp.float32)
    zmat = jnp.zeros((NPAD, D), jnp.float32)

    degp = _sc_degrees(src2d, dst2d, zrow)
    degs = jnp.moveaxis(degp.reshape(4, NPAD), 0, 1)

    t1 = _tc1(fpad, degs, W1)
    agg1 = _sc_aggregate(t1, src2d, dst2d, zmat)
    t2 = _tc2(agg1, degs, b1.reshape(1, D), g1.reshape(1, D),
              be1.reshape(1, D), W2)
    agg2 = _sc_aggregate(t2, src2d, dst2d, zmat)
    out = _tc3(agg2, degs, b2.reshape(1, D), g2.reshape(1, D),
               be2.reshape(1, D))
    return out[:N]
